# baseline (speedup 1.0000x reference)
import functools

import jax
import jax.numpy as jnp
from jax import lax
from jax.experimental import pallas as pl
from jax.experimental.pallas import tpu as pltpu
from jax.experimental.pallas import tpu_sc as plsc

N_ITEMS = 16384
D = 64
NF = 5
N_STYLES = 1000
N_BREWERS = 100000
EPS9 = 9e-5

NC = 2
NS = 16
NW = NC * NS
CHUNK = 4096
NCHUNK = N_ITEMS // CHUNK
BLK = 2048
NBLK = N_ITEMS // BLK
ROW_PAD = 100096


def _gather_body(bt_h, st_h, bid_h, sid_h, h1_h,
                 brow_v, srow_v, bid_v, sid_v, out_v,
                 sem_row, sem_ids, sem_out):
    wid = lax.axis_index("c") * NS + lax.axis_index("s")

    def fetch_ids(ci, buf):
        base = ci * CHUNK
        pltpu.async_copy(bid_h.at[pl.ds(base, CHUNK)], bid_v.at[buf], sem_ids)
        pltpu.async_copy(sid_h.at[pl.ds(base, CHUNK)], sid_v.at[buf], sem_ids)

    def drain_ids(ci, buf):
        pltpu.make_async_copy(bid_h.at[pl.ds(ci * CHUNK, CHUNK)], bid_v.at[buf], sem_ids).wait()
        pltpu.make_async_copy(sid_h.at[pl.ds(ci * CHUNK, CHUNK)], sid_v.at[buf], sem_ids).wait()

    for p in range(2):
        dim = p * NW + wid
        rcp = pltpu.async_copy(bt_h.at[dim], brow_v, sem_row)
        scp = pltpu.async_copy(st_h.at[dim], srow_v, sem_row)
        if p == 0:
            fetch_ids(0, 0)
        rcp.wait()
        scp.wait()

        def chunk_body(ci, carry):
            buf = lax.rem(ci, 2)
            drain_ids(ci, buf)

            @pl.when(ci + 1 < NCHUNK)
            def _():
                fetch_ids(ci + 1, 1 - buf)

            @pl.when(ci >= 2)
            def _():
                pltpu.make_async_copy(
                    out_v.at[buf], h1_h.at[dim, pl.ds((ci - 2) * CHUNK, CHUNK)],
                    sem_out).wait()

            for v0 in range(0, CHUNK // 16, 8):
                bis = [bid_v[buf, pl.ds(16 * (v0 + u), 16)] for u in range(8)]
                sis = [sid_v[buf, pl.ds(16 * (v0 + u), 16)] for u in range(8)]
                gb = [plsc.load_gather(brow_v, [bi]) for bi in bis]
                gs = [plsc.load_gather(srow_v, [si]) for si in sis]
                for u in range(8):
                    out_v[buf, pl.ds(16 * (v0 + u), 16)] = gb[u] + gs[u]
            pltpu.async_copy(out_v.at[buf], h1_h.at[dim, pl.ds(ci * CHUNK, CHUNK)], sem_out)
            return carry

        lax.fori_loop(0, NCHUNK, chunk_body, 0)
        for ci in range(NCHUNK - 2, NCHUNK):
            pltpu.make_async_copy(
                out_v.at[ci % 2], h1_h.at[dim, pl.ds(ci * CHUNK, CHUNK)],
                sem_out).wait()
        if p == 0:
            fetch_ids(0, 0)


def _ln_tc_body(h1_ref, ct_ref, wbg_ref, out_ref):
    wbg = wbg_ref[...]
    w = wbg[:, 0:NF]
    b = wbg[:, NF:NF + 1]
    g = wbg[:, NF + 1:NF + 2]
    be = wbg[:, NF + 2:NF + 3]
    c = jax.lax.dot_general(
        w, ct_ref[...], (((1,), (0,)), ((), ())),
        preferred_element_type=jnp.float32)
    h = h1_ref[...] + c + b
    mean = jnp.mean(h, axis=0, keepdims=True)
    var = jnp.mean(h * h, axis=0, keepdims=True) - mean * mean
    inv = jax.lax.rsqrt(var + EPS9)
    out_ref[...] = (h - mean) * inv * g + be


def kernel(style_ids, brewer_ids, cont_feats, style_table, brewer_table, W, b, gamma, beta):
    bt = brewer_table.T
    st = style_table.T
    ct = cont_feats.T

    mesh = plsc.VectorSubcoreMesh(core_axis_name="c", subcore_axis_name="s")
    params = pltpu.CompilerParams(needs_layout_passes=False, use_tc_tiling_on_sc=True)

    gather = pl.kernel(
        _gather_body,
        out_type=jax.ShapeDtypeStruct((D, N_ITEMS), jnp.float32),
        mesh=mesh,
        compiler_params=params,
        scratch_types=[
            pltpu.VMEM((N_BREWERS,), jnp.float32),
            pltpu.VMEM((N_STYLES,), jnp.float32),
            pltpu.VMEM((2, CHUNK), jnp.int32),
            pltpu.VMEM((2, CHUNK), jnp.int32),
            pltpu.VMEM((2, CHUNK), jnp.float32),
            pltpu.SemaphoreType.DMA,
            pltpu.SemaphoreType.DMA,
            pltpu.SemaphoreType.DMA,
        ],
    )
    h1 = gather(bt, st, brewer_ids, style_ids)

    wbg = jnp.concatenate(
        [W, b[:, None], gamma[:, None], beta[:, None]], axis=1)
    out_t = pl.pallas_call(
        _ln_tc_body,
        out_shape=jax.ShapeDtypeStruct((D, N_ITEMS), jnp.float32),
        grid=(NBLK,),
        in_specs=[
            pl.BlockSpec((D, BLK), lambda i: (0, i)),
            pl.BlockSpec((NF, BLK), lambda i: (0, i)),
            pl.BlockSpec((D, NF + 3), lambda i: (0, 0)),
        ],
        out_specs=pl.BlockSpec((D, BLK), lambda i: (0, i)),
    )(h1, ct, wbg)
    return out_t.T


# speedup vs baseline: 1.0055x; 1.0055x over previous
import functools

import jax
import jax.numpy as jnp
from jax import lax
from jax.experimental import pallas as pl
from jax.experimental.pallas import tpu as pltpu
from jax.experimental.pallas import tpu_sc as plsc

N_ITEMS = 16384
D = 64
NF = 5
N_STYLES = 1000
N_BREWERS = 100000
EPS9 = 9e-5

NC = 2
NS = 16
NW = NC * NS
CHUNK = 4096
NCHUNK = N_ITEMS // CHUNK
BLK = 2048
NBLK = N_ITEMS // BLK


def _gather_body(bt_h, st_h, bid_h, sid_h, h1_h,
                 brow_v, srow_v, bid_v, sid_v, out_v,
                 sem_row, sem_ids, sem_out):
    wid = lax.axis_index("c") * NS + lax.axis_index("s")

    def fetch_ids(ci, buf):
        base = ci * CHUNK
        pltpu.async_copy(bid_h.at[pl.ds(base, CHUNK)], bid_v.at[buf], sem_ids)
        pltpu.async_copy(sid_h.at[pl.ds(base, CHUNK)], sid_v.at[buf], sem_ids)

    def drain_ids(ci, buf):
        pltpu.make_async_copy(bid_h.at[pl.ds(ci * CHUNK, CHUNK)], bid_v.at[buf], sem_ids).wait()
        pltpu.make_async_copy(sid_h.at[pl.ds(ci * CHUNK, CHUNK)], sid_v.at[buf], sem_ids).wait()

    for p in range(2):
        dim = p * NW + wid
        rcp = pltpu.async_copy(bt_h.at[dim], brow_v, sem_row)
        scp = pltpu.async_copy(st_h.at[dim], srow_v, sem_row)
        if p == 1:
            for ci in range(NCHUNK - 2, NCHUNK):
                pltpu.make_async_copy(
                    out_v.at[ci % 2], h1_h.at[wid, pl.ds(ci * CHUNK, CHUNK)],
                    sem_out).wait()
        fetch_ids(0, 0)
        rcp.wait()
        scp.wait()

        def chunk_body(ci, carry):
            buf = lax.rem(ci, 2)
            drain_ids(ci, buf)

            @pl.when(ci + 1 < NCHUNK)
            def _():
                fetch_ids(ci + 1, 1 - buf)

            @pl.when(ci >= 2)
            def _():
                pltpu.make_async_copy(
                    out_v.at[buf], h1_h.at[dim, pl.ds((ci - 2) * CHUNK, CHUNK)],
                    sem_out).wait()

            for v0 in range(0, CHUNK // 16, 8):
                bis = [bid_v[buf, pl.ds(16 * (v0 + u), 16)] for u in range(8)]
                sis = [sid_v[buf, pl.ds(16 * (v0 + u), 16)] for u in range(8)]
                gb = [plsc.load_gather(brow_v, [bi]) for bi in bis]
                gs = [plsc.load_gather(srow_v, [si]) for si in sis]
                for u in range(8):
                    out_v[buf, pl.ds(16 * (v0 + u), 16)] = gb[u] + gs[u]
            pltpu.async_copy(out_v.at[buf], h1_h.at[dim, pl.ds(ci * CHUNK, CHUNK)], sem_out)
            return carry

        lax.fori_loop(0, NCHUNK, chunk_body, 0)
    for ci in range(NCHUNK - 2, NCHUNK):
        pltpu.make_async_copy(
            out_v.at[ci % 2], h1_h.at[NW + wid, pl.ds(ci * CHUNK, CHUNK)],
            sem_out).wait()


def _ln_tc_body(h1_ref, ct_ref, wbg_ref, out_ref):
    wbg = wbg_ref[...]
    w = wbg[:, 0:NF]
    b = wbg[:, NF:NF + 1]
    g = wbg[:, NF + 1:NF + 2]
    be = wbg[:, NF + 2:NF + 3]
    c = jax.lax.dot_general(
        w, ct_ref[...], (((1,), (0,)), ((), ())),
        preferred_element_type=jnp.float32)
    h = h1_ref[...] + c + b
    mean = jnp.mean(h, axis=0, keepdims=True)
    var = jnp.mean(h * h, axis=0, keepdims=True) - mean * mean
    inv = jax.lax.rsqrt(var + EPS9)
    out_ref[...] = (h - mean) * inv * g + be


def kernel(style_ids, brewer_ids, cont_feats, style_table, brewer_table, W, b, gamma, beta):
    bt = brewer_table.T
    st = style_table.T
    ct = cont_feats.T

    mesh = plsc.VectorSubcoreMesh(core_axis_name="c", subcore_axis_name="s")
    params = pltpu.CompilerParams(needs_layout_passes=False, use_tc_tiling_on_sc=True)

    gather = pl.kernel(
        _gather_body,
        out_type=jax.ShapeDtypeStruct((D, N_ITEMS), jnp.float32),
        mesh=mesh,
        compiler_params=params,
        scratch_types=[
            pltpu.VMEM((N_BREWERS,), jnp.float32),
            pltpu.VMEM((N_STYLES,), jnp.float32),
            pltpu.VMEM((2, CHUNK), jnp.int32),
            pltpu.VMEM((2, CHUNK), jnp.int32),
            pltpu.VMEM((2, CHUNK), jnp.float32),
            pltpu.SemaphoreType.DMA,
            pltpu.SemaphoreType.DMA,
            pltpu.SemaphoreType.DMA,
        ],
    )
    h1 = gather(bt, st, brewer_ids, style_ids)

    wbg = jnp.concatenate(
        [W, b[:, None], gamma[:, None], beta[:, None]], axis=1)
    out_t = pl.pallas_call(
        _ln_tc_body,
        out_shape=jax.ShapeDtypeStruct((D, N_ITEMS), jnp.float32),
        grid=(NBLK,),
        in_specs=[
            pl.BlockSpec((D, BLK), lambda i: (0, i)),
            pl.BlockSpec((NF, BLK), lambda i: (0, i)),
            pl.BlockSpec((D, NF + 3), lambda i: (0, 0)),
        ],
        out_specs=pl.BlockSpec((D, BLK), lambda i: (0, i)),
    )(h1, ct, wbg)
    return out_t.T


# speedup vs baseline: 1.0412x; 1.0355x over previous
import functools

import jax
import jax.numpy as jnp
from jax import lax
from jax.experimental import pallas as pl
from jax.experimental.pallas import tpu as pltpu
from jax.experimental.pallas import tpu_sc as plsc

N_ITEMS = 16384
D = 64
NF = 5
N_STYLES = 1000
N_BREWERS = 100000
EPS9 = 9e-5

NC = 2
NS = 16
NW = NC * NS
CHUNK = 4096
NCHUNK = N_ITEMS // CHUNK
BLK = 4096
NBLK = N_ITEMS // BLK


def _gather_body(bt_h, st_h, bid_h, sid_h, h1_h,
                 brow_v, srow_v, bid_v, sid_v, out_v,
                 sem_row, sem_ids, sem_out):
    wid = lax.axis_index("c") * NS + lax.axis_index("s")

    def fetch_ids(ci, buf):
        base = ci * CHUNK
        pltpu.async_copy(bid_h.at[pl.ds(base, CHUNK)], bid_v.at[buf], sem_ids)
        pltpu.async_copy(sid_h.at[pl.ds(base, CHUNK)], sid_v.at[buf], sem_ids)

    def drain_ids(ci, buf):
        pltpu.make_async_copy(bid_h.at[pl.ds(ci * CHUNK, CHUNK)], bid_v.at[buf], sem_ids).wait()
        pltpu.make_async_copy(sid_h.at[pl.ds(ci * CHUNK, CHUNK)], sid_v.at[buf], sem_ids).wait()

    for p in range(2):
        dim = p * NW + wid
        rcp = pltpu.async_copy(bt_h.at[dim], brow_v, sem_row)
        scp = pltpu.async_copy(st_h.at[dim], srow_v, sem_row)
        if p == 1:
            for ci in range(NCHUNK - 2, NCHUNK):
                pltpu.make_async_copy(
                    out_v.at[ci % 2], h1_h.at[wid, pl.ds(ci * CHUNK, CHUNK)],
                    sem_out).wait()
        fetch_ids(0, 0)
        rcp.wait()
        scp.wait()

        def chunk_body(ci, carry):
            buf = lax.rem(ci, 2)
            drain_ids(ci, buf)

            @pl.when(ci + 1 < NCHUNK)
            def _():
                fetch_ids(ci + 1, 1 - buf)

            @pl.when(ci >= 2)
            def _():
                pltpu.make_async_copy(
                    out_v.at[buf], h1_h.at[dim, pl.ds((ci - 2) * CHUNK, CHUNK)],
                    sem_out).wait()

            for v0 in range(0, CHUNK // 16, 8):
                bis = [bid_v[buf, pl.ds(16 * (v0 + u), 16)] for u in range(8)]
                sis = [sid_v[buf, pl.ds(16 * (v0 + u), 16)] for u in range(8)]
                gb = [plsc.load_gather(brow_v, [bi]) for bi in bis]
                gs = [plsc.load_gather(srow_v, [si]) for si in sis]
                for u in range(8):
                    out_v[buf, pl.ds(16 * (v0 + u), 16)] = gb[u] + gs[u]
            pltpu.async_copy(out_v.at[buf], h1_h.at[dim, pl.ds(ci * CHUNK, CHUNK)], sem_out)
            return carry

        lax.fori_loop(0, NCHUNK, chunk_body, 0)
    for ci in range(NCHUNK - 2, NCHUNK):
        pltpu.make_async_copy(
            out_v.at[ci % 2], h1_h.at[NW + wid, pl.ds(ci * CHUNK, CHUNK)],
            sem_out).wait()


def _ln_tc_body(h1_ref, ct_ref, wbg_ref, out_ref):
    wbg = wbg_ref[...]
    w = wbg[:, 0:NF]
    b = wbg[:, NF:NF + 1]
    g = wbg[:, NF + 1:NF + 2]
    be = wbg[:, NF + 2:NF + 3]
    c = jax.lax.dot_general(
        w, ct_ref[...], (((1,), (0,)), ((), ())),
        preferred_element_type=jnp.float32)
    h = h1_ref[...] + c + b
    mean = jnp.mean(h, axis=0, keepdims=True)
    var = jnp.mean(h * h, axis=0, keepdims=True) - mean * mean
    inv = jax.lax.rsqrt(var + EPS9)
    out_ref[...] = (h - mean) * inv * g + be


def kernel(style_ids, brewer_ids, cont_feats, style_table, brewer_table, W, b, gamma, beta):
    bt = brewer_table.T
    st = style_table.T
    ct = cont_feats.T

    mesh = plsc.VectorSubcoreMesh(core_axis_name="c", subcore_axis_name="s")
    params = pltpu.CompilerParams(needs_layout_passes=False, use_tc_tiling_on_sc=True)

    gather = pl.kernel(
        _gather_body,
        out_type=jax.ShapeDtypeStruct((D, N_ITEMS), jnp.float32),
        mesh=mesh,
        compiler_params=params,
        scratch_types=[
            pltpu.VMEM((N_BREWERS,), jnp.float32),
            pltpu.VMEM((N_STYLES,), jnp.float32),
            pltpu.VMEM((2, CHUNK), jnp.int32),
            pltpu.VMEM((2, CHUNK), jnp.int32),
            pltpu.VMEM((2, CHUNK), jnp.float32),
            pltpu.SemaphoreType.DMA,
            pltpu.SemaphoreType.DMA,
            pltpu.SemaphoreType.DMA,
        ],
    )
    h1 = gather(bt, st, brewer_ids, style_ids)

    wbg = jnp.concatenate(
        [W, b[:, None], gamma[:, None], beta[:, None]], axis=1)
    out_t = pl.pallas_call(
        _ln_tc_body,
        out_shape=jax.ShapeDtypeStruct((D, N_ITEMS), jnp.float32),
        grid=(NBLK,),
        in_specs=[
            pl.BlockSpec((D, BLK), lambda i: (0, i)),
            pl.BlockSpec((NF, BLK), lambda i: (0, i)),
            pl.BlockSpec((D, NF + 3), lambda i: (0, 0)),
        ],
        out_specs=pl.BlockSpec((D, BLK), lambda i: (0, i)),
    )(h1, ct, wbg)
    return out_t.T


# speedup vs baseline: 1.0633x; 1.0212x over previous
import functools

import jax
import jax.numpy as jnp
from jax import lax
from jax.experimental import pallas as pl
from jax.experimental.pallas import tpu as pltpu
from jax.experimental.pallas import tpu_sc as plsc

N_ITEMS = 16384
D = 64
NF = 5
N_STYLES = 1000
N_BREWERS = 100000
EPS9 = 9e-5

NC = 2
NS = 16
NW = NC * NS
CHUNK = 4096
NCHUNK = N_ITEMS // CHUNK
BLK = 8192
NBLK = N_ITEMS // BLK


def _gather_body(bt_h, st_h, bid_h, sid_h, h1_h,
                 brow_v, srow_v, bid_v, sid_v, out_v,
                 sem_row, sem_ids, sem_out):
    wid = lax.axis_index("c") * NS + lax.axis_index("s")

    def fetch_ids(ci, buf):
        base = ci * CHUNK
        pltpu.async_copy(bid_h.at[pl.ds(base, CHUNK)], bid_v.at[buf], sem_ids)
        pltpu.async_copy(sid_h.at[pl.ds(base, CHUNK)], sid_v.at[buf], sem_ids)

    def drain_ids(ci, buf):
        pltpu.make_async_copy(bid_h.at[pl.ds(ci * CHUNK, CHUNK)], bid_v.at[buf], sem_ids).wait()
        pltpu.make_async_copy(sid_h.at[pl.ds(ci * CHUNK, CHUNK)], sid_v.at[buf], sem_ids).wait()

    for p in range(2):
        dim = p * NW + wid
        rcp = pltpu.async_copy(bt_h.at[dim], brow_v, sem_row)
        scp = pltpu.async_copy(st_h.at[dim], srow_v, sem_row)
        if p == 1:
            for ci in range(NCHUNK - 2, NCHUNK):
                pltpu.make_async_copy(
                    out_v.at[ci % 2], h1_h.at[wid, pl.ds(ci * CHUNK, CHUNK)],
                    sem_out).wait()
        fetch_ids(0, 0)
        rcp.wait()
        scp.wait()

        def chunk_body(ci, carry):
            buf = lax.rem(ci, 2)
            drain_ids(ci, buf)

            @pl.when(ci + 1 < NCHUNK)
            def _():
                fetch_ids(ci + 1, 1 - buf)

            @pl.when(ci >= 2)
            def _():
                pltpu.make_async_copy(
                    out_v.at[buf], h1_h.at[dim, pl.ds((ci - 2) * CHUNK, CHUNK)],
                    sem_out).wait()

            for v0 in range(0, CHUNK // 16, 8):
                bis = [bid_v[buf, pl.ds(16 * (v0 + u), 16)] for u in range(8)]
                sis = [sid_v[buf, pl.ds(16 * (v0 + u), 16)] for u in range(8)]
                gb = [plsc.load_gather(brow_v, [bi]) for bi in bis]
                gs = [plsc.load_gather(srow_v, [si]) for si in sis]
                for u in range(8):
                    out_v[buf, pl.ds(16 * (v0 + u), 16)] = gb[u] + gs[u]
            pltpu.async_copy(out_v.at[buf], h1_h.at[dim, pl.ds(ci * CHUNK, CHUNK)], sem_out)
            return carry

        lax.fori_loop(0, NCHUNK, chunk_body, 0)
    for ci in range(NCHUNK - 2, NCHUNK):
        pltpu.make_async_copy(
            out_v.at[ci % 2], h1_h.at[NW + wid, pl.ds(ci * CHUNK, CHUNK)],
            sem_out).wait()


def _ln_tc_body(h1_ref, ct_ref, wbg_ref, out_ref):
    wbg = wbg_ref[...]
    w = wbg[:, 0:NF]
    b = wbg[:, NF:NF + 1]
    g = wbg[:, NF + 1:NF + 2]
    be = wbg[:, NF + 2:NF + 3]
    c = jax.lax.dot_general(
        w, ct_ref[...], (((1,), (0,)), ((), ())),
        preferred_element_type=jnp.float32)
    h = h1_ref[...] + c + b
    mean = jnp.mean(h, axis=0, keepdims=True)
    var = jnp.mean(h * h, axis=0, keepdims=True) - mean * mean
    inv = jax.lax.rsqrt(var + EPS9)
    out_ref[...] = (h - mean) * inv * g + be


def kernel(style_ids, brewer_ids, cont_feats, style_table, brewer_table, W, b, gamma, beta):
    bt = brewer_table.T
    st = style_table.T
    ct = cont_feats.T

    mesh = plsc.VectorSubcoreMesh(core_axis_name="c", subcore_axis_name="s")
    params = pltpu.CompilerParams(needs_layout_passes=False, use_tc_tiling_on_sc=True)

    gather = pl.kernel(
        _gather_body,
        out_type=jax.ShapeDtypeStruct((D, N_ITEMS), jnp.float32),
        mesh=mesh,
        compiler_params=params,
        scratch_types=[
            pltpu.VMEM((N_BREWERS,), jnp.float32),
            pltpu.VMEM((N_STYLES,), jnp.float32),
            pltpu.VMEM((2, CHUNK), jnp.int32),
            pltpu.VMEM((2, CHUNK), jnp.int32),
            pltpu.VMEM((2, CHUNK), jnp.float32),
            pltpu.SemaphoreType.DMA,
            pltpu.SemaphoreType.DMA,
            pltpu.SemaphoreType.DMA,
        ],
    )
    h1 = gather(bt, st, brewer_ids, style_ids)

    wbg = jnp.concatenate(
        [W, b[:, None], gamma[:, None], beta[:, None]], axis=1)
    out_t = pl.pallas_call(
        _ln_tc_body,
        out_shape=jax.ShapeDtypeStruct((D, N_ITEMS), jnp.float32),
        grid=(NBLK,),
        in_specs=[
            pl.BlockSpec((D, BLK), lambda i: (0, i)),
            pl.BlockSpec((NF, BLK), lambda i: (0, i)),
            pl.BlockSpec((D, NF + 3), lambda i: (0, 0)),
        ],
        out_specs=pl.BlockSpec((D, BLK), lambda i: (0, i)),
    )(h1, ct, wbg)
    return out_t.T
